# baseline (speedup 1.0000x reference)
import functools

import jax
import jax.numpy as jnp
from jax.experimental import pallas as pl


def _pass1_body(adj_ref, xb_ref, w1_ref, b1_ref, w2_ref, g2_ref, adjb_ref):
    ab = adj_ref[...].astype(jnp.bfloat16)
    adjb_ref[...] = ab
    t = jnp.dot(ab, xb_ref[...], preferred_element_type=jnp.float32)
    h = jnp.maximum(
        jnp.dot(t, w1_ref[...], preferred_element_type=jnp.float32) + b1_ref[...],
        0.0,
    )
    g2_ref[...] = jnp.dot(h, w2_ref[...], preferred_element_type=jnp.float32).astype(
        jnp.bfloat16
    )


def _pass2_body(adjb_ref, g2_ref, b2_ref, w3_ref, g3_ref):
    t = jnp.dot(adjb_ref[...], g2_ref[...], preferred_element_type=jnp.float32)
    h = jnp.maximum(t + b2_ref[...], 0.0)
    g3_ref[...] = jnp.dot(h, w3_ref[...], preferred_element_type=jnp.float32).astype(
        jnp.bfloat16
    )


def _pass3_body(adjb_ref, g3_ref, b3_ref, out_ref):
    out_ref[...] = (
        jnp.dot(adjb_ref[...], g3_ref[...], preferred_element_type=jnp.float32)
        + b3_ref[...]
    )


def _block_rows(n: int) -> int:
    for bm in (80, 16, 8):
        if n % bm == 0:
            return bm
    return n


@jax.jit
def kernel(x, adj, labels, W1, b1, W2, b2, W3, b3):
    del labels
    n, nfeat = x.shape
    nhid = W1.shape[1]
    ncls = W3.shape[1]
    bm = _block_rows(n)
    grid = (n // bm,)

    xb = x.astype(jnp.bfloat16)
    b1r = b1.reshape(1, nhid)
    b2r = b2.reshape(1, nhid)
    b3r = b3.reshape(1, ncls)

    full = lambda shape: pl.BlockSpec(shape, lambda i: (0, 0))
    rows = lambda cols: pl.BlockSpec((bm, cols), lambda i: (i, 0))

    g2, adjb = pl.pallas_call(
        _pass1_body,
        grid=grid,
        in_specs=[
            rows(n),
            full((n, nfeat)),
            full((nfeat, nhid)),
            full((1, nhid)),
            full((nhid, nhid)),
        ],
        out_specs=[rows(nhid), rows(n)],
        out_shape=[
            jax.ShapeDtypeStruct((n, nhid), jnp.bfloat16),
            jax.ShapeDtypeStruct((n, n), jnp.bfloat16),
        ],
    )(adj, xb, W1, b1r, W2)

    g3 = pl.pallas_call(
        _pass2_body,
        grid=grid,
        in_specs=[rows(n), full((n, nhid)), full((1, nhid)), full((nhid, ncls))],
        out_specs=rows(ncls),
        out_shape=jax.ShapeDtypeStruct((n, ncls), jnp.bfloat16),
    )(adjb, g2, b2r, W3)

    out = pl.pallas_call(
        _pass3_body,
        grid=grid,
        in_specs=[rows(n), full((n, ncls)), full((1, ncls))],
        out_specs=rows(ncls),
        out_shape=jax.ShapeDtypeStruct((n, ncls), jnp.float32),
    )(adjb, g3, b3r)
    return out


# speedup vs baseline: 1.1575x; 1.1575x over previous
import jax
import jax.numpy as jnp
from jax.experimental import pallas as pl

_BM = 256


def _pass1_body(adj_ref, xb_ref, w1_ref, b1_ref, w2_ref, g2_ref, adjq_ref):
    a = adj_ref[...]
    adjq_ref[...] = jnp.round(a * 254.0 - 127.0).astype(jnp.int8)
    t = jnp.dot(a.astype(jnp.bfloat16), xb_ref[...],
                preferred_element_type=jnp.float32)
    h = jnp.maximum(
        jnp.dot(t, w1_ref[...], preferred_element_type=jnp.float32) + b1_ref[...],
        0.0,
    )
    g2_ref[...] = jnp.dot(h, w2_ref[...], preferred_element_type=jnp.float32)


def _quant_body(g_ref, ghi_ref, glo_ref, corr_ref, scale_ref):
    g = g_ref[...]
    amax = jnp.maximum(jnp.max(jnp.abs(g), axis=0, keepdims=True), 1e-30)
    inv = 127.0 / amax
    r = g * inv
    hi = jnp.round(r)
    lo = jnp.round((r - hi) * 254.0)
    ghi_ref[...] = hi.astype(jnp.int8)
    glo_ref[...] = lo.astype(jnp.int8)
    colsum = jnp.sum(hi, axis=0, keepdims=True) + jnp.sum(lo, axis=0, keepdims=True) * (1.0 / 254.0)
    corr_ref[...] = 127.0 * colsum
    scale_ref[...] = amax * (1.0 / (127.0 * 254.0))


def _pass2_body(adjq_ref, ghi_ref, glo_ref, corr_ref, scale_ref, b2_ref, w3_ref,
                g3_ref):
    aq = adjq_ref[...]
    dhi = jnp.dot(aq, ghi_ref[...], preferred_element_type=jnp.int32)
    dlo = jnp.dot(aq, glo_ref[...], preferred_element_type=jnp.int32)
    t = (dhi.astype(jnp.float32) + dlo.astype(jnp.float32) * (1.0 / 254.0)
         + corr_ref[...]) * scale_ref[...]
    h = jnp.maximum(t + b2_ref[...], 0.0)
    g3_ref[...] = jnp.dot(h, w3_ref[...], preferred_element_type=jnp.float32)


def _pass3_body(adjq_ref, ghi_ref, glo_ref, corr_ref, scale_ref, b3_ref, out_ref):
    aq = adjq_ref[...]
    dhi = jnp.dot(aq, ghi_ref[...], preferred_element_type=jnp.int32)
    dlo = jnp.dot(aq, glo_ref[...], preferred_element_type=jnp.int32)
    out_ref[...] = (dhi.astype(jnp.float32) + dlo.astype(jnp.float32) * (1.0 / 254.0)
                    + corr_ref[...]) * scale_ref[...] + b3_ref[...]


def _quantize(g):
    n, d = g.shape
    full = lambda shape: pl.BlockSpec(shape, lambda: (0, 0))
    return pl.pallas_call(
        _quant_body,
        in_specs=[full((n, d))],
        out_specs=[full((n, d)), full((n, d)), full((1, d)), full((1, d))],
        out_shape=[
            jax.ShapeDtypeStruct((n, d), jnp.int8),
            jax.ShapeDtypeStruct((n, d), jnp.int8),
            jax.ShapeDtypeStruct((1, d), jnp.float32),
            jax.ShapeDtypeStruct((1, d), jnp.float32),
        ],
    )(g)


@jax.jit
def kernel(x, adj, labels, W1, b1, W2, b2, W3, b3):
    del labels
    n, nfeat = x.shape
    nhid = W1.shape[1]
    ncls = W3.shape[1]
    bm = min(_BM, n)
    grid = (pl.cdiv(n, bm),)

    xb = x.astype(jnp.bfloat16)
    b1r = b1.reshape(1, nhid)
    b2r = b2.reshape(1, nhid)
    b3r = b3.reshape(1, ncls)

    full = lambda shape: pl.BlockSpec(shape, lambda i: (0, 0))
    rows = lambda cols: pl.BlockSpec((bm, cols), lambda i: (i, 0))

    g2, adjq = pl.pallas_call(
        _pass1_body,
        grid=grid,
        in_specs=[
            rows(n),
            full((n, nfeat)),
            full((nfeat, nhid)),
            full((1, nhid)),
            full((nhid, nhid)),
        ],
        out_specs=[rows(nhid), rows(n)],
        out_shape=[
            jax.ShapeDtypeStruct((n, nhid), jnp.float32),
            jax.ShapeDtypeStruct((n, n), jnp.int8),
        ],
    )(adj, xb, W1, b1r, W2)

    g2hi, g2lo, c2, s2 = _quantize(g2)
    g3 = pl.pallas_call(
        _pass2_body,
        grid=grid,
        in_specs=[rows(n), full((n, nhid)), full((n, nhid)), full((1, nhid)),
                  full((1, nhid)), full((1, nhid)), full((nhid, ncls))],
        out_specs=rows(ncls),
        out_shape=jax.ShapeDtypeStruct((n, ncls), jnp.float32),
    )(adjq, g2hi, g2lo, c2, s2, b2r, W3)

    g3hi, g3lo, c3, s3 = _quantize(g3)
    out = pl.pallas_call(
        _pass3_body,
        grid=grid,
        in_specs=[rows(n), full((n, ncls)), full((n, ncls)), full((1, ncls)),
                  full((1, ncls)), full((1, ncls))],
        out_specs=rows(ncls),
        out_shape=jax.ShapeDtypeStruct((n, ncls), jnp.float32),
    )(adjq, g3hi, g3lo, c3, s3, b3r)
    return out


# speedup vs baseline: 1.4726x; 1.2722x over previous
import functools

import jax
import jax.numpy as jnp
from jax.experimental import pallas as pl


def _pass1_body(adj_ref, xb_ref, w1_ref, b1_ref, w2_ref, g2_ref, adjb_ref):
    ab = adj_ref[...].astype(jnp.bfloat16)
    adjb_ref[...] = ab
    t = jnp.dot(ab, xb_ref[...], preferred_element_type=jnp.float32)
    h = jnp.maximum(
        jnp.dot(t, w1_ref[...], preferred_element_type=jnp.float32) + b1_ref[...],
        0.0,
    )
    g2_ref[...] = jnp.dot(h, w2_ref[...], preferred_element_type=jnp.float32).astype(
        jnp.bfloat16
    )


def _pass2_body(adjb_ref, g2_ref, b2_ref, w3_ref, g3_ref):
    t = jnp.dot(adjb_ref[...], g2_ref[...], preferred_element_type=jnp.float32)
    h = jnp.maximum(t + b2_ref[...], 0.0)
    g3_ref[...] = jnp.dot(h, w3_ref[...], preferred_element_type=jnp.float32).astype(
        jnp.bfloat16
    )


def _pass3_body(adjb_ref, g3_ref, b3_ref, out_ref):
    out_ref[...] = (
        jnp.dot(adjb_ref[...], g3_ref[...], preferred_element_type=jnp.float32)
        + b3_ref[...]
    )


_BM1 = 400
_BM2 = 800
_BM3 = 800


@jax.jit
def kernel(x, adj, labels, W1, b1, W2, b2, W3, b3):
    del labels
    n, nfeat = x.shape
    nhid = W1.shape[1]
    ncls = W3.shape[1]
    bm1, bm2, bm3 = (min(b, n) for b in (_BM1, _BM2, _BM3))

    xb = x.astype(jnp.bfloat16)
    b1r = b1.reshape(1, nhid)
    b2r = b2.reshape(1, nhid)
    b3r = b3.reshape(1, ncls)

    full = lambda shape: pl.BlockSpec(shape, lambda i: (0, 0))
    rows = lambda bm, cols: pl.BlockSpec((bm, cols), lambda i: (i, 0))

    g2, adjb = pl.pallas_call(
        _pass1_body,
        grid=(pl.cdiv(n, bm1),),
        in_specs=[
            rows(bm1, n),
            full((n, nfeat)),
            full((nfeat, nhid)),
            full((1, nhid)),
            full((nhid, nhid)),
        ],
        out_specs=[rows(bm1, nhid), rows(bm1, n)],
        out_shape=[
            jax.ShapeDtypeStruct((n, nhid), jnp.bfloat16),
            jax.ShapeDtypeStruct((n, n), jnp.bfloat16),
        ],
    )(adj, xb, W1, b1r, W2)

    g3 = pl.pallas_call(
        _pass2_body,
        grid=(pl.cdiv(n, bm2),),
        in_specs=[rows(bm2, n), full((n, nhid)), full((1, nhid)),
                  full((nhid, ncls))],
        out_specs=rows(bm2, ncls),
        out_shape=jax.ShapeDtypeStruct((n, ncls), jnp.bfloat16),
    )(adjb, g2, b2r, W3)

    out = pl.pallas_call(
        _pass3_body,
        grid=(pl.cdiv(n, bm3),),
        in_specs=[rows(bm3, n), full((n, ncls)), full((1, ncls))],
        out_specs=rows(bm3, ncls),
        out_shape=jax.ShapeDtypeStruct((n, ncls), jnp.float32),
    )(adjb, g3, b3r)
    return out


# speedup vs baseline: 1.4850x; 1.0084x over previous
import functools

import jax
import jax.numpy as jnp
from jax.experimental import pallas as pl


def _pass1_body(adj_ref, xb_ref, w1_ref, b1_ref, w2_ref, g2_ref, adjb_ref):
    ab = adj_ref[...].astype(jnp.bfloat16)
    adjb_ref[...] = ab
    t = jnp.dot(ab, xb_ref[...], preferred_element_type=jnp.float32)
    h = jnp.maximum(
        jnp.dot(t, w1_ref[...], preferred_element_type=jnp.float32) + b1_ref[...],
        0.0,
    )
    g2_ref[...] = jnp.dot(h, w2_ref[...], preferred_element_type=jnp.float32).astype(
        jnp.bfloat16
    )


def _pass2_body(adjb_ref, g2_ref, b2_ref, w3_ref, g3_ref):
    t = jnp.dot(adjb_ref[...], g2_ref[...], preferred_element_type=jnp.float32)
    h = jnp.maximum(t + b2_ref[...], 0.0)
    g3_ref[...] = jnp.dot(h, w3_ref[...], preferred_element_type=jnp.float32).astype(
        jnp.bfloat16
    )


def _pass3_body(adjb_ref, g3_ref, b3_ref, out_ref):
    out_ref[...] = (
        jnp.dot(adjb_ref[...], g3_ref[...], preferred_element_type=jnp.float32)
        + b3_ref[...]
    )


_BM1 = 400
_BM2 = 1120
_BM3 = 1120


@jax.jit
def kernel(x, adj, labels, W1, b1, W2, b2, W3, b3):
    del labels
    n, nfeat = x.shape
    nhid = W1.shape[1]
    ncls = W3.shape[1]
    bm1, bm2, bm3 = (min(b, n) for b in (_BM1, _BM2, _BM3))

    xb = x.astype(jnp.bfloat16)
    b1r = b1.reshape(1, nhid)
    b2r = b2.reshape(1, nhid)
    b3r = b3.reshape(1, ncls)

    full = lambda shape: pl.BlockSpec(shape, lambda i: (0, 0))
    rows = lambda bm, cols: pl.BlockSpec((bm, cols), lambda i: (i, 0))

    g2, adjb = pl.pallas_call(
        _pass1_body,
        grid=(pl.cdiv(n, bm1),),
        in_specs=[
            rows(bm1, n),
            full((n, nfeat)),
            full((nfeat, nhid)),
            full((1, nhid)),
            full((nhid, nhid)),
        ],
        out_specs=[rows(bm1, nhid), rows(bm1, n)],
        out_shape=[
            jax.ShapeDtypeStruct((n, nhid), jnp.bfloat16),
            jax.ShapeDtypeStruct((n, n), jnp.bfloat16),
        ],
    )(adj, xb, W1, b1r, W2)

    g3 = pl.pallas_call(
        _pass2_body,
        grid=(pl.cdiv(n, bm2),),
        in_specs=[rows(bm2, n), full((n, nhid)), full((1, nhid)),
                  full((nhid, ncls))],
        out_specs=rows(bm2, ncls),
        out_shape=jax.ShapeDtypeStruct((n, ncls), jnp.bfloat16),
    )(adjb, g2, b2r, W3)

    out = pl.pallas_call(
        _pass3_body,
        grid=(pl.cdiv(n, bm3),),
        in_specs=[rows(bm3, n), full((n, ncls)), full((1, ncls))],
        out_specs=rows(bm3, ncls),
        out_shape=jax.ShapeDtypeStruct((n, ncls), jnp.float32),
    )(adjb, g3, b3r)
    return out
